# baseline (speedup 1.0000x reference)
import jax
import jax.numpy as jnp
from jax import lax
from jax.experimental import pallas as pl
from jax.experimental.pallas import tpu as pltpu
from jax.experimental.pallas import tpu_sc as plsc

BATCH = 16384
N_FIELDS = 26
VOCAB = 100000
DIM = 50

_NC = 2
_NS = 16
_NW = _NC * _NS
_L = 16

TOTAL_ROWS = BATCH * N_FIELDS
ROWS_PER_W = TOTAL_ROWS // _NW
WORDS_PER_W = ROWS_PER_W * DIM

PAT_W = 400
PAT_ROWS = PAT_W // DIM
PAT_VECS = PAT_W // _L

CHUNK_W = 20800
CHUNK_ROWS = CHUNK_W // DIM
CHUNK_BLOCKS = CHUNK_W // PAT_W
N_CHUNKS = WORDS_PER_W // CHUNK_W

_PAT_A = [(j * _L) // DIM for j in range(PAT_VECS)]
_PAT_T = [(_PAT_A[j] + 1) * DIM - j * _L for j in range(PAT_VECS)]


def _embed_kernel(labels_hbm, table_hbm, out_hbm,
                  gidx_v, idx0, idx1, val0, val1, sem0, sem1):
    wid = lax.axis_index("s") * _NC + lax.axis_index("c")
    row_base = wid * ROWS_PER_W
    word_base = row_base * DIM

    pltpu.sync_copy(labels_hbm.at[pl.ds(row_base, ROWS_PER_W)], gidx_v)
    lane = lax.iota(jnp.int32, _L)

    def fix(i, carry):
        sl = pl.ds(i * _L, _L)
        r = row_base + i * _L + lane
        f = lax.rem(r, N_FIELDS)
        gidx_v[sl] = gidx_v[sl] * DIM + f * (VOCAB * DIM)
        return carry

    lax.fori_loop(0, ROWS_PER_W // _L, fix, 0)

    row_off = []
    col = []
    for j in range(PAT_VECS):
        a, t = _PAT_A[j], _PAT_T[j]
        bump = (lane >= t).astype(jnp.int32)
        row_off.append(a + bump)
        col.append((j * _L - a * DIM) + lane - bump * DIM)

    idx_bufs = (idx0, idx1)
    val_bufs = (val0, val1)
    sems = (sem0, sem1)

    def start(k, c):
        def build_block(b, carry):
            rbase = c * CHUNK_ROWS + b * PAT_ROWS
            for j in range(PAT_VECS):
                g = plsc.load_gather(gidx_v, [rbase + row_off[j]])
                idx_bufs[k][pl.ds(b * PAT_W + j * _L, _L)] = g + col[j]
            return carry

        lax.fori_loop(0, CHUNK_BLOCKS, build_block, 0)
        pltpu.async_copy(table_hbm.at[idx_bufs[k]], val_bufs[k], sems[k])

    def drain(k, c):
        pltpu.make_async_copy(
            table_hbm.at[idx_bufs[k]], val_bufs[k], sems[k]).wait()
        pltpu.sync_copy(val_bufs[k],
                        out_hbm.at[pl.ds(word_base + c * CHUNK_W, CHUNK_W)])

    start(0, 0)

    def pair(i, carry):
        g = 2 * i
        start(1, g + 1)
        drain(0, g)
        start(0, g + 2)
        drain(1, g + 1)
        return carry

    lax.fori_loop(0, N_CHUNKS // 2 - 1, pair, 0)
    g_last = N_CHUNKS - 2
    start(1, g_last + 1)
    drain(0, g_last)
    drain(1, g_last + 1)


@jax.jit
def kernel(labels, tables):
    labels_flat = labels.reshape(TOTAL_ROWS)
    table_1d = tables.reshape(N_FIELDS * VOCAB * DIM)

    mesh = plsc.VectorSubcoreMesh(core_axis_name="c", subcore_axis_name="s")
    run = pl.kernel(
        _embed_kernel,
        mesh=mesh,
        out_type=jax.ShapeDtypeStruct((TOTAL_ROWS * DIM,), jnp.float32),
        scratch_types=[
            pltpu.VMEM((ROWS_PER_W,), jnp.int32),
            pltpu.VMEM((CHUNK_W,), jnp.int32),
            pltpu.VMEM((CHUNK_W,), jnp.int32),
            pltpu.VMEM((CHUNK_W,), jnp.float32),
            pltpu.VMEM((CHUNK_W,), jnp.float32),
            pltpu.SemaphoreType.DMA,
            pltpu.SemaphoreType.DMA,
        ],
        compiler_params=pltpu.CompilerParams(
            use_tc_tiling_on_sc=False, needs_layout_passes=False),
    )
    out = run(labels_flat, table_1d)
    return out.reshape(BATCH, N_FIELDS * DIM)


# speedup vs baseline: 1.1593x; 1.1593x over previous
import jax
import jax.numpy as jnp
from jax import lax
from jax.experimental import pallas as pl
from jax.experimental.pallas import tpu as pltpu
from jax.experimental.pallas import tpu_sc as plsc

BATCH = 16384
N_FIELDS = 26
VOCAB = 100000
DIM = 50

_NC = 2
_NS = 16
_NW = _NC * _NS
_L = 16

TOTAL_ROWS = BATCH * N_FIELDS
ROWS_PER_W = TOTAL_ROWS // _NW
WORDS_PER_W = ROWS_PER_W * DIM

GRAN = 16
GRANS_ROW = 4

CHUNK_W = 10400
CHUNK_ROWS = CHUNK_W // DIM
CHUNK_GRANS = CHUNK_ROWS * GRANS_ROW
N_CHUNKS = WORDS_PER_W // CHUNK_W

PAT_W = 400
PAT_ROWS = PAT_W // DIM
PAT_VECS = PAT_W // _L
CHUNK_BLOCKS = CHUNK_W // PAT_W

_PAT_A = [(j * _L) // DIM for j in range(PAT_VECS)]
_PAT_T = [(_PAT_A[j] + 1) * DIM - j * _L for j in range(PAT_VECS)]


def _embed_kernel(labels_hbm, table_hbm, out_hbm,
                  gidx_v, idx0, idx1, stage0, stage1, out0, out1,
                  sem0, sem1):
    wid = lax.axis_index("s") * _NC + lax.axis_index("c")
    row_base = wid * ROWS_PER_W
    word_base = row_base * DIM

    pltpu.sync_copy(labels_hbm.at[pl.ds(row_base, ROWS_PER_W)], gidx_v)
    lane = lax.iota(jnp.int32, _L)

    def fix(i, carry):
        sl = pl.ds(i * _L, _L)
        r = row_base + i * _L + lane
        f = lax.rem(r, N_FIELDS)
        gidx_v[sl] = gidx_v[sl] * DIM + f * (VOCAB * DIM)
        return carry

    lax.fori_loop(0, ROWS_PER_W // _L, fix, 0)

    row_off = []
    col = []
    for j in range(PAT_VECS):
        a, t = _PAT_A[j], _PAT_T[j]
        bump = (lane >= t).astype(jnp.int32)
        row_off.append(a + bump)
        col.append((j * _L - a * DIM) + lane - bump * DIM)

    g_row = lane >> 2
    g_quad = lane & 3

    idx_bufs = (idx0, idx1)
    stage_bufs = (stage0, stage1)
    out_bufs = (out0, out1)
    sems = (sem0, sem1)

    def start(k, c):
        def build(v, carry):
            rbase = c * CHUNK_ROWS + v * 4
            w50 = plsc.load_gather(gidx_v, [rbase + g_row])
            idx_bufs[k][pl.ds(v * _L, _L)] = (w50 >> 4) + g_quad
            return carry

        lax.fori_loop(0, CHUNK_GRANS // _L, build, 0)
        pltpu.async_copy(table_hbm.at[idx_bufs[k]], stage_bufs[k], sems[k])

    def finish(k, c):
        pltpu.make_async_copy(
            table_hbm.at[idx_bufs[k]], stage_bufs[k], sems[k]).wait()

        def compact(b, carry):
            for j in range(PAT_VECS):
                r_loc = b * PAT_ROWS + row_off[j]
                w50 = plsc.load_gather(gidx_v, [c * CHUNK_ROWS + r_loc])
                s = (r_loc << 6) + (w50 & 15) + col[j]
                val = plsc.load_gather(stage_bufs[k], [s >> 4, s & 15])
                out_bufs[k][pl.ds(b * PAT_W + j * _L, _L)] = val
            return carry

        lax.fori_loop(0, CHUNK_BLOCKS, compact, 0)
        pltpu.sync_copy(out_bufs[k],
                        out_hbm.at[pl.ds(word_base + c * CHUNK_W, CHUNK_W)])

    start(0, 0)

    def pair(i, carry):
        g = 2 * i
        start(1, g + 1)
        finish(0, g)
        start(0, g + 2)
        finish(1, g + 1)
        return carry

    lax.fori_loop(0, N_CHUNKS // 2 - 1, pair, 0)
    g_last = N_CHUNKS - 2
    start(1, g_last + 1)
    finish(0, g_last)
    finish(1, g_last + 1)


@jax.jit
def kernel(labels, tables):
    labels_flat = labels.reshape(TOTAL_ROWS)
    table_g = tables.reshape(N_FIELDS * VOCAB * DIM // GRAN, GRAN)

    mesh = plsc.VectorSubcoreMesh(core_axis_name="c", subcore_axis_name="s")
    run = pl.kernel(
        _embed_kernel,
        mesh=mesh,
        out_type=jax.ShapeDtypeStruct((TOTAL_ROWS * DIM,), jnp.float32),
        scratch_types=[
            pltpu.VMEM((ROWS_PER_W,), jnp.int32),
            pltpu.VMEM((CHUNK_GRANS,), jnp.int32),
            pltpu.VMEM((CHUNK_GRANS,), jnp.int32),
            pltpu.VMEM((CHUNK_GRANS, GRAN), jnp.float32),
            pltpu.VMEM((CHUNK_GRANS, GRAN), jnp.float32),
            pltpu.VMEM((CHUNK_W,), jnp.float32),
            pltpu.VMEM((CHUNK_W,), jnp.float32),
            pltpu.SemaphoreType.DMA,
            pltpu.SemaphoreType.DMA,
        ],
        compiler_params=pltpu.CompilerParams(
            use_tc_tiling_on_sc=False, needs_layout_passes=False),
    )
    out = run(labels_flat, table_g)
    return out.reshape(BATCH, N_FIELDS * DIM)
